# baseline (speedup 1.0000x reference)
import jax
import jax.numpy as jnp
from jax import lax
from jax.experimental import pallas as pl
from jax.experimental.pallas import tpu as pltpu
from jax.experimental.pallas import tpu_sc as plsc

_N_BINS = 15
_L = 16
_UNROLL = 7
_NCH = 2


def _ece_partials(conf, pred, lab, *, count, num_cores, num_subcores):
    nw = num_cores * num_subcores
    n = count
    assert n % _L == 0
    total_vec = n // _L
    base_vec = total_vec // nw
    nbig = total_vec - base_vec * nw
    nv = base_vec + (1 if nbig else 0)
    short_elems = base_vec * _L

    cv = -(-nv // (_NCH * _UNROLL)) * _UNROLL
    cb = cv * _L
    last_small = short_elems - (_NCH - 1) * cb
    assert 0 < last_small <= cb and last_small % _L == 0
    zfill = (cb - last_small) // _L

    def body(conf_hbm, pred_hbm, lab_hbm, out_hbm,
             conf_v0, pred_v0, lab_v0, conf_v1, pred_v1, lab_v1,
             acc_v, buf_v, sem):
        slots = ((conf_v0, pred_v0, lab_v0), (conf_v1, pred_v1, lab_v1))
        wid = lax.axis_index("s") * num_cores + lax.axis_index("c")
        base = wid * short_elems + _L * jnp.minimum(wid, nbig)

        zero = jnp.zeros((_L,), jnp.float32)
        lane = lax.iota(jnp.int32, _L)
        tabv = lane.astype(jnp.float32) / jnp.float32(_N_BINS)

        def start_chunk(k):
            cv_, pv_, lv_ = slots[k % 2]
            st = base + k * cb
            if k < _NCH - 1:
                return [
                    pltpu.async_copy(conf_hbm.at[pl.ds(st, cb)], cv_, sem),
                    pltpu.async_copy(pred_hbm.at[pl.ds(st, cb)], pv_, sem),
                    pltpu.async_copy(lab_hbm.at[pl.ds(st, cb)], lv_, sem),
                ]
            for t in range(zfill):
                cv_[pl.ds(last_small + t * _L, _L)] = zero
            cps = [
                pltpu.async_copy(conf_hbm.at[pl.ds(st, last_small)],
                                 cv_.at[pl.ds(0, last_small)], sem),
                pltpu.async_copy(pred_hbm.at[pl.ds(st, last_small)],
                                 pv_.at[pl.ds(0, last_small)], sem),
                pltpu.async_copy(lab_hbm.at[pl.ds(st, last_small)],
                                 lv_.at[pl.ds(0, last_small)], sem),
            ]
            if nbig:
                @pl.when(wid < nbig)
                def _():
                    g = base + short_elems
                    o = last_small
                    pltpu.sync_copy(conf_hbm.at[pl.ds(g, _L)],
                                    cv_.at[pl.ds(o, _L)])
                    pltpu.sync_copy(pred_hbm.at[pl.ds(g, _L)],
                                    pv_.at[pl.ds(o, _L)])
                    pltpu.sync_copy(lab_hbm.at[pl.ds(g, _L)],
                                    lv_.at[pl.ds(o, _L)])
            return cps

        for r in range(_L):
            acc_v[r, :] = zero

        def one(slot, off):
            cv_, pv_, lv_ = slots[slot]
            c = cv_[pl.ds(off, _L)]
            p = pv_[pl.ds(off, _L)]
            l = lv_[pl.ds(off, _L)]
            a = jnp.where(p == l, jnp.float32(1.0), jnp.float32(0.0))
            d = c - a
            ti = (c * jnp.float32(15.0)).astype(jnp.int32)
            lo = jnp.take_along_axis(tabv, ti, axis=0)
            b = ti - (c == lo).astype(jnp.int32)
            plsc.addupdate_scatter(acc_v, [lane, b], d,
                                   mask=c > jnp.float32(0.0))

        cps = start_chunk(0)
        for k in range(_NCH):
            nxt = start_chunk(k + 1) if k + 1 < _NCH else None
            for cp in cps:
                cp.wait()
            slot = k % 2

            @plsc.parallel_loop(0, cb, _L, unroll=_UNROLL)
            def _(off):
                one(slot, off)

            cps = nxt

        tot = acc_v[0, :]
        for r in range(1, _L):
            tot = tot + acc_v[r, :]
        buf_v[...] = tot
        pltpu.sync_copy(buf_v, out_hbm.at[wid])

    mesh = plsc.VectorSubcoreMesh(
        core_axis_name="c", subcore_axis_name="s",
        num_cores=num_cores, num_subcores=num_subcores)
    kfn = pl.kernel(
        body,
        out_type=jax.ShapeDtypeStruct((nw, _L), jnp.float32),
        mesh=mesh,
        compiler_params=pltpu.CompilerParams(needs_layout_passes=False),
        scratch_types=[
            pltpu.VMEM((cb,), jnp.float32),
            pltpu.VMEM((cb,), jnp.int32),
            pltpu.VMEM((cb,), jnp.int32),
            pltpu.VMEM((cb,), jnp.float32),
            pltpu.VMEM((cb,), jnp.int32),
            pltpu.VMEM((cb,), jnp.int32),
            pltpu.VMEM((_L, _L), jnp.float32),
            pltpu.VMEM((_L,), jnp.float32),
            pltpu.SemaphoreType.DMA,
        ],
    )
    return kfn(conf, pred, lab)


_TC_BR = 8
_TC_LANES = 1024


def _tc_hist(conf_t, pred_t, lab_t):
    rows = conf_t.shape[0]
    grid = rows // _TC_BR
    assert rows % _TC_BR == 0

    def body(c_ref, p_ref, l_ref, out_ref, acc_ref):
        i = pl.program_id(0)

        @pl.when(i == 0)
        def _():
            acc_ref[...] = jnp.zeros_like(acc_ref)

        c = c_ref[...]
        a = jnp.where(p_ref[...] == l_ref[...],
                      jnp.float32(1.0), jnp.float32(0.0))
        d = c - a
        t = c * jnp.float32(15.0)
        tf = jnp.floor(t)
        bf = tf - (t == tf).astype(jnp.float32)
        bf = jnp.where(bf == jnp.float32(15.0), jnp.float32(14.0), bf)
        for b in range(_N_BINS):
            acc_ref[b] += jnp.where(bf == jnp.float32(b), d, jnp.float32(0.0))

        @pl.when(i == grid - 1)
        def _():
            out_ref[...] = jnp.sum(acc_ref[...], axis=(1, 2))

    blk = lambda: pl.BlockSpec((_TC_BR, _TC_LANES), lambda i: (i, 0))
    return pl.pallas_call(
        body,
        grid=(grid,),
        in_specs=[blk(), blk(), blk()],
        out_specs=pl.BlockSpec((_N_BINS,), lambda i: (0,)),
        out_shape=jax.ShapeDtypeStruct((_N_BINS,), jnp.float32),
        scratch_shapes=[pltpu.VMEM((_N_BINS, _TC_BR, _TC_LANES), jnp.float32)],
    )(conf_t, pred_t, lab_t)


@jax.jit
def kernel(confidences, predictions, labels):
    n = confidences.shape[0]
    blk_elems = _TC_BR * _TC_LANES
    t_count = (n // 2) // blk_elems * blk_elems
    s_count = n - t_count
    assert s_count % _L == 0

    parts = _ece_partials(confidences, predictions, labels, count=s_count,
                          num_cores=2, num_subcores=16)

    rows = t_count // _TC_LANES
    conf_t = confidences[s_count:].reshape(rows, _TC_LANES)
    pred_t = predictions[s_count:].reshape(rows, _TC_LANES)
    lab_t = labels[s_count:].reshape(rows, _TC_LANES)
    tc = _tc_hist(conf_t, pred_t, lab_t)

    s = parts.sum(axis=0)
    bins = s[:_N_BINS] + tc
    ece = (jnp.abs(bins[:_N_BINS - 1]).sum()
           + jnp.abs(bins[_N_BINS - 1] + s[_N_BINS])) / jnp.float32(n)
    return ece.reshape(1)


# speedup vs baseline: 2.0966x; 2.0966x over previous
import jax
import jax.numpy as jnp
from jax import lax
from jax.experimental import pallas as pl
from jax.experimental.pallas import tpu as pltpu
from jax.experimental.pallas import tpu_sc as plsc

_N_BINS = 15
_L = 16
_UNROLL = 7
_NCH = 2


def _ece_partials(conf, pred, lab, *, start, count, num_cores, num_subcores):
    nw = num_cores * num_subcores
    n = count
    assert n % _L == 0 and start % _L == 0
    total_vec = n // _L
    base_vec = total_vec // nw
    nbig = total_vec - base_vec * nw
    nv = base_vec + (1 if nbig else 0)
    short_elems = base_vec * _L

    cv = -(-nv // (_NCH * _UNROLL)) * _UNROLL
    cb = cv * _L
    last_small = short_elems - (_NCH - 1) * cb
    assert 0 < last_small <= cb and last_small % _L == 0
    zfill = (cb - last_small) // _L

    def body(conf_hbm, pred_hbm, lab_hbm, out_hbm,
             conf_v0, pred_v0, lab_v0, conf_v1, pred_v1, lab_v1,
             acc_v, buf_v, sem):
        slots = ((conf_v0, pred_v0, lab_v0), (conf_v1, pred_v1, lab_v1))
        wid = lax.axis_index("s") * num_cores + lax.axis_index("c")
        base = start + wid * short_elems + _L * jnp.minimum(wid, nbig)

        zero = jnp.zeros((_L,), jnp.float32)
        lane = lax.iota(jnp.int32, _L)
        tabv = lane.astype(jnp.float32) / jnp.float32(_N_BINS)

        def start_chunk(k):
            cv_, pv_, lv_ = slots[k % 2]
            st = base + k * cb
            if k < _NCH - 1:
                return [
                    pltpu.async_copy(conf_hbm.at[pl.ds(st, cb)], cv_, sem),
                    pltpu.async_copy(pred_hbm.at[pl.ds(st, cb)], pv_, sem),
                    pltpu.async_copy(lab_hbm.at[pl.ds(st, cb)], lv_, sem),
                ]
            for t in range(zfill):
                cv_[pl.ds(last_small + t * _L, _L)] = zero
            cps = [
                pltpu.async_copy(conf_hbm.at[pl.ds(st, last_small)],
                                 cv_.at[pl.ds(0, last_small)], sem),
                pltpu.async_copy(pred_hbm.at[pl.ds(st, last_small)],
                                 pv_.at[pl.ds(0, last_small)], sem),
                pltpu.async_copy(lab_hbm.at[pl.ds(st, last_small)],
                                 lv_.at[pl.ds(0, last_small)], sem),
            ]
            if nbig:
                @pl.when(wid < nbig)
                def _():
                    g = base + short_elems
                    o = last_small
                    pltpu.sync_copy(conf_hbm.at[pl.ds(g, _L)],
                                    cv_.at[pl.ds(o, _L)])
                    pltpu.sync_copy(pred_hbm.at[pl.ds(g, _L)],
                                    pv_.at[pl.ds(o, _L)])
                    pltpu.sync_copy(lab_hbm.at[pl.ds(g, _L)],
                                    lv_.at[pl.ds(o, _L)])
            return cps

        for r in range(_L):
            acc_v[r, :] = zero

        def one(slot, off):
            cv_, pv_, lv_ = slots[slot]
            c = cv_[pl.ds(off, _L)]
            p = pv_[pl.ds(off, _L)]
            l = lv_[pl.ds(off, _L)]
            a = jnp.where(p == l, jnp.float32(1.0), jnp.float32(0.0))
            d = c - a
            ti = (c * jnp.float32(15.0)).astype(jnp.int32)
            lo = jnp.take_along_axis(tabv, ti, axis=0)
            b = ti - (c == lo).astype(jnp.int32)
            plsc.addupdate_scatter(acc_v, [lane, b], d,
                                   mask=c > jnp.float32(0.0))

        cps = start_chunk(0)
        for k in range(_NCH):
            nxt = start_chunk(k + 1) if k + 1 < _NCH else None
            for cp in cps:
                cp.wait()
            slot = k % 2

            @plsc.parallel_loop(0, cb, _L, unroll=_UNROLL)
            def _(off):
                one(slot, off)

            cps = nxt

        tot = acc_v[0, :]
        for r in range(1, _L):
            tot = tot + acc_v[r, :]
        buf_v[...] = tot
        pltpu.sync_copy(buf_v, out_hbm.at[wid])

    mesh = plsc.VectorSubcoreMesh(
        core_axis_name="c", subcore_axis_name="s",
        num_cores=num_cores, num_subcores=num_subcores)
    kfn = pl.kernel(
        body,
        out_type=jax.ShapeDtypeStruct((nw, _L), jnp.float32),
        mesh=mesh,
        compiler_params=pltpu.CompilerParams(needs_layout_passes=False),
        scratch_types=[
            pltpu.VMEM((cb,), jnp.float32),
            pltpu.VMEM((cb,), jnp.int32),
            pltpu.VMEM((cb,), jnp.int32),
            pltpu.VMEM((cb,), jnp.float32),
            pltpu.VMEM((cb,), jnp.int32),
            pltpu.VMEM((cb,), jnp.int32),
            pltpu.VMEM((_L, _L), jnp.float32),
            pltpu.VMEM((_L,), jnp.float32),
            pltpu.SemaphoreType.DMA,
        ],
    )
    return kfn(conf, pred, lab)


_TC_CHUNK = 8192
_TC_NCHUNK = 15
_TC_BLK = _TC_CHUNK * _TC_NCHUNK
_TC_GRID = 4


def _tc_hist(conf, pred, lab):
    def body(c_ref, p_ref, l_ref, out_ref, acc_ref):
        i = pl.program_id(0)
        acc = [jnp.zeros((1024,), jnp.float32) for _ in range(_N_BINS)]
        for j in range(_TC_NCHUNK):
            sl = pl.ds(j * _TC_CHUNK, _TC_CHUNK)
            c = c_ref[sl]
            a = jnp.where(p_ref[sl] == l_ref[sl],
                          jnp.float32(1.0), jnp.float32(0.0))
            d = c - a
            t = c * jnp.float32(15.0)
            tf = jnp.floor(t)
            bf = tf - (t == tf).astype(jnp.float32)
            bf = jnp.where(bf == jnp.float32(15.0), jnp.float32(14.0), bf)
            for b in range(_N_BINS):
                m = jnp.where(bf == jnp.float32(b), d, jnp.float32(0.0))
                r = acc[b]
                for q in range(_TC_CHUNK // 1024):
                    r = r + m[q * 1024:(q + 1) * 1024]
                acc[b] = r

        st = jnp.stack(acc)

        @pl.when(i == 0)
        def _():
            acc_ref[...] = st

        @pl.when(i > 0)
        def _():
            acc_ref[...] += st

        @pl.when(i == _TC_GRID - 1)
        def _():
            out_ref[...] = acc_ref[...].sum(axis=1)

    blk = lambda: pl.BlockSpec((_TC_BLK,), lambda i: (i,))
    return pl.pallas_call(
        body,
        grid=(_TC_GRID,),
        in_specs=[blk(), blk(), blk()],
        out_specs=pl.BlockSpec((_N_BINS,), lambda i: (0,)),
        out_shape=jax.ShapeDtypeStruct((_N_BINS,), jnp.float32),
        scratch_shapes=[pltpu.VMEM((_N_BINS, 1024), jnp.float32)],
    )(conf, pred, lab)


@jax.jit
def kernel(confidences, predictions, labels):
    n = confidences.shape[0]
    t_count = _TC_BLK * _TC_GRID
    s_count = n - t_count
    assert s_count % _L == 0

    parts = _ece_partials(confidences, predictions, labels,
                          start=t_count, count=s_count,
                          num_cores=2, num_subcores=16)
    tc = _tc_hist(confidences, predictions, labels)

    s = parts.sum(axis=0)
    bins = s[:_N_BINS] + tc
    ece = (jnp.abs(bins[:_N_BINS - 1]).sum()
           + jnp.abs(bins[_N_BINS - 1] + s[_N_BINS])) / jnp.float32(n)
    return ece.reshape(1)


# speedup vs baseline: 2.1302x; 1.0160x over previous
import jax
import jax.numpy as jnp
from jax import lax
from jax.experimental import pallas as pl
from jax.experimental.pallas import tpu as pltpu
from jax.experimental.pallas import tpu_sc as plsc

_N_BINS = 15
_L = 16
_UNROLL = 7
_NCH = 1


def _ece_partials(conf, pred, lab, *, start, count, num_cores, num_subcores):
    nw = num_cores * num_subcores
    n = count
    assert n % _L == 0 and start % _L == 0
    total_vec = n // _L
    base_vec = total_vec // nw
    nbig = total_vec - base_vec * nw
    nv = base_vec + (1 if nbig else 0)
    short_elems = base_vec * _L

    cv = -(-nv // (_NCH * _UNROLL)) * _UNROLL
    cb = cv * _L
    last_small = short_elems - (_NCH - 1) * cb
    assert 0 < last_small <= cb and last_small % _L == 0
    zfill = (cb - last_small) // _L

    def body(conf_hbm, pred_hbm, lab_hbm, out_hbm,
             conf_v0, pred_v0, lab_v0, conf_v1, pred_v1, lab_v1,
             acc_v, buf_v, sem):
        slots = ((conf_v0, pred_v0, lab_v0), (conf_v1, pred_v1, lab_v1))
        wid = lax.axis_index("s") * num_cores + lax.axis_index("c")
        base = start + wid * short_elems + _L * jnp.minimum(wid, nbig)

        zero = jnp.zeros((_L,), jnp.float32)
        lane = lax.iota(jnp.int32, _L)
        tabv = lane.astype(jnp.float32) / jnp.float32(_N_BINS)

        def start_chunk(k):
            cv_, pv_, lv_ = slots[k % 2]
            st = base + k * cb
            if k < _NCH - 1:
                return [
                    pltpu.async_copy(conf_hbm.at[pl.ds(st, cb)], cv_, sem),
                    pltpu.async_copy(pred_hbm.at[pl.ds(st, cb)], pv_, sem),
                    pltpu.async_copy(lab_hbm.at[pl.ds(st, cb)], lv_, sem),
                ]
            for t in range(zfill):
                cv_[pl.ds(last_small + t * _L, _L)] = zero
            cps = [
                pltpu.async_copy(conf_hbm.at[pl.ds(st, last_small)],
                                 cv_.at[pl.ds(0, last_small)], sem),
                pltpu.async_copy(pred_hbm.at[pl.ds(st, last_small)],
                                 pv_.at[pl.ds(0, last_small)], sem),
                pltpu.async_copy(lab_hbm.at[pl.ds(st, last_small)],
                                 lv_.at[pl.ds(0, last_small)], sem),
            ]
            if nbig:
                @pl.when(wid < nbig)
                def _():
                    g = base + short_elems
                    o = last_small
                    pltpu.sync_copy(conf_hbm.at[pl.ds(g, _L)],
                                    cv_.at[pl.ds(o, _L)])
                    pltpu.sync_copy(pred_hbm.at[pl.ds(g, _L)],
                                    pv_.at[pl.ds(o, _L)])
                    pltpu.sync_copy(lab_hbm.at[pl.ds(g, _L)],
                                    lv_.at[pl.ds(o, _L)])
            return cps

        for r in range(_L):
            acc_v[r, :] = zero

        def one(slot, off):
            cv_, pv_, lv_ = slots[slot]
            c = cv_[pl.ds(off, _L)]
            p = pv_[pl.ds(off, _L)]
            l = lv_[pl.ds(off, _L)]
            a = jnp.where(p == l, jnp.float32(1.0), jnp.float32(0.0))
            d = c - a
            ti = (c * jnp.float32(15.0)).astype(jnp.int32)
            lo = jnp.take_along_axis(tabv, ti, axis=0)
            b = ti - (c == lo).astype(jnp.int32)
            plsc.addupdate_scatter(acc_v, [lane, b], d,
                                   mask=c > jnp.float32(0.0))

        cps = start_chunk(0)
        for k in range(_NCH):
            nxt = start_chunk(k + 1) if k + 1 < _NCH else None
            for cp in cps:
                cp.wait()
            slot = k % 2

            @plsc.parallel_loop(0, cb, _L, unroll=_UNROLL)
            def _(off):
                one(slot, off)

            cps = nxt

        tot = acc_v[0, :]
        for r in range(1, _L):
            tot = tot + acc_v[r, :]
        buf_v[...] = tot
        pltpu.sync_copy(buf_v, out_hbm.at[wid])

    mesh = plsc.VectorSubcoreMesh(
        core_axis_name="c", subcore_axis_name="s",
        num_cores=num_cores, num_subcores=num_subcores)
    kfn = pl.kernel(
        body,
        out_type=jax.ShapeDtypeStruct((nw, _L), jnp.float32),
        mesh=mesh,
        compiler_params=pltpu.CompilerParams(needs_layout_passes=False),
        scratch_types=[
            pltpu.VMEM((cb,), jnp.float32),
            pltpu.VMEM((cb,), jnp.int32),
            pltpu.VMEM((cb,), jnp.int32),
            pltpu.VMEM((cb,), jnp.float32),
            pltpu.VMEM((cb,), jnp.int32),
            pltpu.VMEM((cb,), jnp.int32),
            pltpu.VMEM((_L, _L), jnp.float32),
            pltpu.VMEM((_L,), jnp.float32),
            pltpu.SemaphoreType.DMA,
        ],
    )
    return kfn(conf, pred, lab)


_TC_CHUNK = 8192
_TC_NCHUNK = 15
_TC_BLK = _TC_CHUNK * _TC_NCHUNK
_TC_GRID = 5


def _tc_hist(conf, pred, lab):
    def body(c_ref, p_ref, l_ref, out_ref, acc_ref):
        i = pl.program_id(0)
        acc = [jnp.zeros((1024,), jnp.float32) for _ in range(_N_BINS)]
        for j in range(_TC_NCHUNK):
            sl = pl.ds(j * _TC_CHUNK, _TC_CHUNK)
            c = c_ref[sl]
            a = jnp.where(p_ref[sl] == l_ref[sl],
                          jnp.float32(1.0), jnp.float32(0.0))
            d = c - a
            t = c * jnp.float32(15.0)
            tf = jnp.floor(t)
            bf = tf - (t == tf).astype(jnp.float32)
            bf = jnp.where(bf == jnp.float32(15.0), jnp.float32(14.0), bf)
            for b in range(_N_BINS):
                m = jnp.where(bf == jnp.float32(b), d, jnp.float32(0.0))
                r = acc[b]
                for q in range(_TC_CHUNK // 1024):
                    r = r + m[q * 1024:(q + 1) * 1024]
                acc[b] = r

        st = jnp.stack(acc)

        @pl.when(i == 0)
        def _():
            acc_ref[...] = st

        @pl.when(i > 0)
        def _():
            acc_ref[...] += st

        @pl.when(i == _TC_GRID - 1)
        def _():
            out_ref[...] = acc_ref[...].sum(axis=1)

    blk = lambda: pl.BlockSpec((_TC_BLK,), lambda i: (i,))
    return pl.pallas_call(
        body,
        grid=(_TC_GRID,),
        in_specs=[blk(), blk(), blk()],
        out_specs=pl.BlockSpec((_N_BINS,), lambda i: (0,)),
        out_shape=jax.ShapeDtypeStruct((_N_BINS,), jnp.float32),
        scratch_shapes=[pltpu.VMEM((_N_BINS, 1024), jnp.float32)],
    )(conf, pred, lab)


@jax.jit
def kernel(confidences, predictions, labels):
    n = confidences.shape[0]
    t_count = _TC_BLK * _TC_GRID
    s_count = n - t_count
    assert s_count % _L == 0

    parts = _ece_partials(confidences, predictions, labels,
                          start=t_count, count=s_count,
                          num_cores=2, num_subcores=16)
    tc = _tc_hist(confidences, predictions, labels)

    s = parts.sum(axis=0)
    bins = s[:_N_BINS] + tc
    ece = (jnp.abs(bins[:_N_BINS - 1]).sum()
           + jnp.abs(bins[_N_BINS - 1] + s[_N_BINS])) / jnp.float32(n)
    return ece.reshape(1)


# speedup vs baseline: 2.4639x; 1.1567x over previous
import jax
import jax.numpy as jnp
from jax import lax
from jax.experimental import pallas as pl
from jax.experimental.pallas import tpu as pltpu
from jax.experimental.pallas import tpu_sc as plsc

_N_BINS = 15
_L = 16
_UNROLL = 7
_NCH = 1


def _ece_partials(conf, pred, lab, *, start, count, num_cores, num_subcores):
    nw = num_cores * num_subcores
    n = count
    assert n % _L == 0 and start % _L == 0
    total_vec = n // _L
    base_vec = total_vec // nw
    nbig = total_vec - base_vec * nw
    nv = base_vec + (1 if nbig else 0)
    short_elems = base_vec * _L

    cv = -(-nv // (_NCH * _UNROLL)) * _UNROLL
    cb = cv * _L
    last_small = short_elems - (_NCH - 1) * cb
    assert 0 < last_small <= cb and last_small % _L == 0
    zfill = (cb - last_small) // _L

    def body(conf_hbm, pred_hbm, lab_hbm, out_hbm,
             conf_v0, pred_v0, lab_v0, conf_v1, pred_v1, lab_v1,
             acc_v, buf_v, sem):
        slots = ((conf_v0, pred_v0, lab_v0), (conf_v1, pred_v1, lab_v1))
        wid = lax.axis_index("s") * num_cores + lax.axis_index("c")
        base = start + wid * short_elems + _L * jnp.minimum(wid, nbig)

        zero = jnp.zeros((_L,), jnp.float32)
        lane = lax.iota(jnp.int32, _L)
        tabv = lane.astype(jnp.float32) / jnp.float32(_N_BINS)

        def start_chunk(k):
            cv_, pv_, lv_ = slots[k % 2]
            st = base + k * cb
            if k < _NCH - 1:
                return [
                    pltpu.async_copy(conf_hbm.at[pl.ds(st, cb)], cv_, sem),
                    pltpu.async_copy(pred_hbm.at[pl.ds(st, cb)], pv_, sem),
                    pltpu.async_copy(lab_hbm.at[pl.ds(st, cb)], lv_, sem),
                ]
            for t in range(zfill):
                cv_[pl.ds(last_small + t * _L, _L)] = zero
            cps = [
                pltpu.async_copy(conf_hbm.at[pl.ds(st, last_small)],
                                 cv_.at[pl.ds(0, last_small)], sem),
                pltpu.async_copy(pred_hbm.at[pl.ds(st, last_small)],
                                 pv_.at[pl.ds(0, last_small)], sem),
                pltpu.async_copy(lab_hbm.at[pl.ds(st, last_small)],
                                 lv_.at[pl.ds(0, last_small)], sem),
            ]
            if nbig:
                @pl.when(wid < nbig)
                def _():
                    g = base + short_elems
                    o = last_small
                    pltpu.sync_copy(conf_hbm.at[pl.ds(g, _L)],
                                    cv_.at[pl.ds(o, _L)])
                    pltpu.sync_copy(pred_hbm.at[pl.ds(g, _L)],
                                    pv_.at[pl.ds(o, _L)])
                    pltpu.sync_copy(lab_hbm.at[pl.ds(g, _L)],
                                    lv_.at[pl.ds(o, _L)])
            return cps

        for r in range(_L):
            acc_v[r, :] = zero

        def one(slot, off):
            cv_, pv_, lv_ = slots[slot]
            c = cv_[pl.ds(off, _L)]
            p = pv_[pl.ds(off, _L)]
            l = lv_[pl.ds(off, _L)]
            a = jnp.where(p == l, jnp.float32(1.0), jnp.float32(0.0))
            d = c - a
            ti = (c * jnp.float32(15.0)).astype(jnp.int32)
            lo = jnp.take_along_axis(tabv, ti, axis=0)
            b = ti - (c == lo).astype(jnp.int32)
            plsc.addupdate_scatter(acc_v, [lane, b], d,
                                   mask=c > jnp.float32(0.0))

        cps = start_chunk(0)
        for k in range(_NCH):
            nxt = start_chunk(k + 1) if k + 1 < _NCH else None
            for cp in cps:
                cp.wait()
            slot = k % 2

            @plsc.parallel_loop(0, cb, _L, unroll=_UNROLL)
            def _(off):
                one(slot, off)

            cps = nxt

        tot = acc_v[0, :]
        for r in range(1, _L):
            tot = tot + acc_v[r, :]
        buf_v[...] = tot
        pltpu.sync_copy(buf_v, out_hbm.at[wid])

    mesh = plsc.VectorSubcoreMesh(
        core_axis_name="c", subcore_axis_name="s",
        num_cores=num_cores, num_subcores=num_subcores)
    kfn = pl.kernel(
        body,
        out_type=jax.ShapeDtypeStruct((nw, _L), jnp.float32),
        mesh=mesh,
        compiler_params=pltpu.CompilerParams(needs_layout_passes=False),
        scratch_types=[
            pltpu.VMEM((cb,), jnp.float32),
            pltpu.VMEM((cb,), jnp.int32),
            pltpu.VMEM((cb,), jnp.int32),
            pltpu.VMEM((cb,), jnp.float32),
            pltpu.VMEM((cb,), jnp.int32),
            pltpu.VMEM((cb,), jnp.int32),
            pltpu.VMEM((_L, _L), jnp.float32),
            pltpu.VMEM((_L,), jnp.float32),
            pltpu.SemaphoreType.DMA,
        ],
    )
    return kfn(conf, pred, lab)


_TC_CHUNK = 8192
_TC_NCHUNK = 15
_TC_BLK = _TC_CHUNK * _TC_NCHUNK
_TC_GRID = 5


def _tc_hist(conf, pred, lab):
    def body(c_ref, p_ref, l_ref, out_ref, acc_ref):
        i = pl.program_id(0)
        acc = [jnp.zeros((1024,), jnp.float32) for _ in range(_N_BINS)]
        for j in range(_TC_NCHUNK):
            sl = pl.ds(j * _TC_CHUNK, _TC_CHUNK)
            c = c_ref[sl]
            a = jnp.where(p_ref[sl] == l_ref[sl],
                          jnp.float32(1.0), jnp.float32(0.0))
            d = c - a
            t = c * jnp.float32(15.0)
            tf = jnp.floor(t)
            bf = tf - (t == tf).astype(jnp.float32)
            bf = jnp.where(bf == jnp.float32(15.0), jnp.float32(14.0), bf)
            for b in range(_N_BINS):
                m = jnp.where(bf == jnp.float32(b), d, jnp.float32(0.0))
                r = acc[b]
                for q in range(_TC_CHUNK // 1024):
                    r = r + m[q * 1024:(q + 1) * 1024]
                acc[b] = r

        st = jnp.stack(acc)

        @pl.when(i == 0)
        def _():
            acc_ref[...] = st

        @pl.when(i > 0)
        def _():
            acc_ref[...] += st

        @pl.when(i == _TC_GRID - 1)
        def _():
            out_ref[...] = acc_ref[...].sum(axis=1)

    blk = lambda: pl.BlockSpec((_TC_BLK,), lambda i: (i,))
    return pl.pallas_call(
        body,
        grid=(_TC_GRID,),
        in_specs=[blk(), blk(), blk()],
        out_specs=pl.BlockSpec((_N_BINS,), lambda i: (0,)),
        out_shape=jax.ShapeDtypeStruct((_N_BINS,), jnp.float32),
        scratch_shapes=[pltpu.VMEM((_N_BINS, 1024), jnp.float32)],
    )(conf, pred, lab)


def _finalize(parts, tc, n):
    def body(p_ref, t_ref, o_ref):
        s = jnp.sum(p_ref[...], axis=0)
        bins = s[:_N_BINS] + t_ref[...]
        e = (jnp.abs(bins[:_N_BINS - 1]).sum()
             + jnp.abs(bins[_N_BINS - 1] + s[_N_BINS])) / jnp.float32(n)
        o_ref[...] = e.reshape(1)

    return pl.pallas_call(
        body,
        out_shape=jax.ShapeDtypeStruct((1,), jnp.float32),
    )(parts, tc)


@jax.jit
def kernel(confidences, predictions, labels):
    n = confidences.shape[0]
    t_count = _TC_BLK * _TC_GRID
    s_count = n - t_count
    assert s_count % _L == 0

    parts = _ece_partials(confidences, predictions, labels,
                          start=t_count, count=s_count,
                          num_cores=2, num_subcores=16)
    tc = _tc_hist(confidences, predictions, labels)
    return _finalize(parts, tc, n)


# speedup vs baseline: 2.5336x; 1.0283x over previous
import jax
import jax.numpy as jnp
from jax import lax
from jax.experimental import pallas as pl
from jax.experimental.pallas import tpu as pltpu
from jax.experimental.pallas import tpu_sc as plsc

_N_BINS = 15
_L = 16
_UNROLL = 7
_NCH = 1


def _ece_partials(conf, pred, lab, *, start, count, num_cores, num_subcores):
    nw = num_cores * num_subcores
    n = count
    assert n % _L == 0 and start % _L == 0
    total_vec = n // _L
    base_vec = total_vec // nw
    nbig = total_vec - base_vec * nw
    nv = base_vec + (1 if nbig else 0)
    short_elems = base_vec * _L

    cv = -(-nv // (_NCH * _UNROLL)) * _UNROLL
    cb = cv * _L
    last_small = short_elems - (_NCH - 1) * cb
    assert 0 < last_small <= cb and last_small % _L == 0
    zfill = (cb - last_small) // _L

    def body(conf_hbm, pred_hbm, lab_hbm, out_hbm,
             conf_v0, pred_v0, lab_v0, conf_v1, pred_v1, lab_v1,
             acc_v, buf_v, sem):
        slots = ((conf_v0, pred_v0, lab_v0), (conf_v1, pred_v1, lab_v1))
        wid = lax.axis_index("s") * num_cores + lax.axis_index("c")
        base = start + wid * short_elems + _L * jnp.minimum(wid, nbig)

        zero = jnp.zeros((_L,), jnp.float32)
        lane = lax.iota(jnp.int32, _L)
        tabv = lane.astype(jnp.float32) / jnp.float32(_N_BINS)

        def start_chunk(k):
            cv_, pv_, lv_ = slots[k % 2]
            st = base + k * cb
            if k < _NCH - 1:
                return [
                    pltpu.async_copy(conf_hbm.at[pl.ds(st, cb)], cv_, sem),
                    pltpu.async_copy(pred_hbm.at[pl.ds(st, cb)], pv_, sem),
                    pltpu.async_copy(lab_hbm.at[pl.ds(st, cb)], lv_, sem),
                ]
            for t in range(zfill):
                cv_[pl.ds(last_small + t * _L, _L)] = zero
            cps = [
                pltpu.async_copy(conf_hbm.at[pl.ds(st, last_small)],
                                 cv_.at[pl.ds(0, last_small)], sem),
                pltpu.async_copy(pred_hbm.at[pl.ds(st, last_small)],
                                 pv_.at[pl.ds(0, last_small)], sem),
                pltpu.async_copy(lab_hbm.at[pl.ds(st, last_small)],
                                 lv_.at[pl.ds(0, last_small)], sem),
            ]
            if nbig:
                @pl.when(wid < nbig)
                def _():
                    g = base + short_elems
                    o = last_small
                    pltpu.sync_copy(conf_hbm.at[pl.ds(g, _L)],
                                    cv_.at[pl.ds(o, _L)])
                    pltpu.sync_copy(pred_hbm.at[pl.ds(g, _L)],
                                    pv_.at[pl.ds(o, _L)])
                    pltpu.sync_copy(lab_hbm.at[pl.ds(g, _L)],
                                    lv_.at[pl.ds(o, _L)])
            return cps

        for r in range(_L):
            acc_v[r, :] = zero

        def one(slot, off):
            cv_, pv_, lv_ = slots[slot]
            c = cv_[pl.ds(off, _L)]
            p = pv_[pl.ds(off, _L)]
            l = lv_[pl.ds(off, _L)]
            a = jnp.where(p == l, jnp.float32(1.0), jnp.float32(0.0))
            d = c - a
            ti = (c * jnp.float32(15.0)).astype(jnp.int32)
            lo = jnp.take_along_axis(tabv, ti, axis=0)
            b = ti - (c == lo).astype(jnp.int32)
            plsc.addupdate_scatter(acc_v, [lane, b], d,
                                   mask=c > jnp.float32(0.0))

        cps = start_chunk(0)
        for k in range(_NCH):
            nxt = start_chunk(k + 1) if k + 1 < _NCH else None
            for cp in cps:
                cp.wait()
            slot = k % 2

            @plsc.parallel_loop(0, cb, _L, unroll=_UNROLL)
            def _(off):
                one(slot, off)

            cps = nxt

        tot = acc_v[0, :]
        for r in range(1, _L):
            tot = tot + acc_v[r, :]
        buf_v[...] = tot
        pltpu.sync_copy(buf_v, out_hbm.at[wid])

    mesh = plsc.VectorSubcoreMesh(
        core_axis_name="c", subcore_axis_name="s",
        num_cores=num_cores, num_subcores=num_subcores)
    kfn = pl.kernel(
        body,
        out_type=jax.ShapeDtypeStruct((nw, _L), jnp.float32),
        mesh=mesh,
        compiler_params=pltpu.CompilerParams(needs_layout_passes=False),
        scratch_types=[
            pltpu.VMEM((cb,), jnp.float32),
            pltpu.VMEM((cb,), jnp.int32),
            pltpu.VMEM((cb,), jnp.int32),
            pltpu.VMEM((cb,), jnp.float32),
            pltpu.VMEM((cb,), jnp.int32),
            pltpu.VMEM((cb,), jnp.int32),
            pltpu.VMEM((_L, _L), jnp.float32),
            pltpu.VMEM((_L,), jnp.float32),
            pltpu.SemaphoreType.DMA,
        ],
    )
    return kfn(conf, pred, lab)


_TC_CHUNK = 8192
_TC_NCHUNK = 15
_TC_BLK = _TC_CHUNK * _TC_NCHUNK
_TC_GRID = 6


def _tc_hist(conf, pred, lab):
    def body(c_ref, p_ref, l_ref, out_ref, acc_ref):
        i = pl.program_id(0)
        acc = [jnp.zeros((1024,), jnp.float32) for _ in range(_N_BINS)]
        for j in range(_TC_NCHUNK):
            sl = pl.ds(j * _TC_CHUNK, _TC_CHUNK)
            c = c_ref[sl]
            a = jnp.where(p_ref[sl] == l_ref[sl],
                          jnp.float32(1.0), jnp.float32(0.0))
            d = c - a
            t = c * jnp.float32(15.0)
            tf = jnp.floor(t)
            bf = tf - (t == tf).astype(jnp.float32)
            bf = jnp.where(bf == jnp.float32(15.0), jnp.float32(14.0), bf)
            for b in range(_N_BINS):
                m = jnp.where(bf == jnp.float32(b), d, jnp.float32(0.0))
                r = acc[b]
                for q in range(_TC_CHUNK // 1024):
                    r = r + m[q * 1024:(q + 1) * 1024]
                acc[b] = r

        st = jnp.stack(acc)

        @pl.when(i == 0)
        def _():
            acc_ref[...] = st

        @pl.when(i > 0)
        def _():
            acc_ref[...] += st

        @pl.when(i == _TC_GRID - 1)
        def _():
            out_ref[...] = acc_ref[...].sum(axis=1)

    blk = lambda: pl.BlockSpec((_TC_BLK,), lambda i: (i,))
    return pl.pallas_call(
        body,
        grid=(_TC_GRID,),
        in_specs=[blk(), blk(), blk()],
        out_specs=pl.BlockSpec((_N_BINS,), lambda i: (0,)),
        out_shape=jax.ShapeDtypeStruct((_N_BINS,), jnp.float32),
        scratch_shapes=[pltpu.VMEM((_N_BINS, 1024), jnp.float32)],
    )(conf, pred, lab)


def _finalize(parts, tc, n):
    def body(p_ref, t_ref, o_ref):
        s = jnp.sum(p_ref[...], axis=0)
        bins = s[:_N_BINS] + t_ref[...]
        e = (jnp.abs(bins[:_N_BINS - 1]).sum()
             + jnp.abs(bins[_N_BINS - 1] + s[_N_BINS])) / jnp.float32(n)
        o_ref[...] = e.reshape(1)

    return pl.pallas_call(
        body,
        out_shape=jax.ShapeDtypeStruct((1,), jnp.float32),
    )(parts, tc)


@jax.jit
def kernel(confidences, predictions, labels):
    n = confidences.shape[0]
    t_count = _TC_BLK * _TC_GRID
    s_count = n - t_count
    assert s_count % _L == 0

    parts = _ece_partials(confidences, predictions, labels,
                          start=t_count, count=s_count,
                          num_cores=2, num_subcores=16)
    tc = _tc_hist(confidences, predictions, labels)
    return _finalize(parts, tc, n)
